# baseline (speedup 1.0000x reference)
import functools

import jax
import jax.numpy as jnp
from jax import lax
from jax.experimental import pallas as pl
from jax.experimental.pallas import tpu as pltpu
from jax.experimental.pallas import tpu_sc as plsc

NC = 2
NS = 16
NW = NC * NS

N = 10000
E = 320000
D = 128

NPAD = 10240
RPS = NPAD // NS
K = 128
CHUNKS = 2560
EPAD = CHUNKS * K
PPC = 40
FAST_P = 4
SLOW_P = 0
FAST_CPT = FAST_P * PPC
SLOW_CPT = SLOW_P * PPC
assert NS * (FAST_CPT + SLOW_CPT) == CHUNKS

_MESH = plsc.VectorSubcoreMesh(
    core_axis_name="c", subcore_axis_name="s", num_cores=NC, num_subcores=NS)
_SC_PARAMS = pltpu.CompilerParams(needs_layout_passes=False)


def _zero_rows(rows):
  z16 = jnp.zeros((16,), jnp.float32)

  def body(r, _):
    for k in range(D // 16):
      rows[r, pl.ds(k * 16, 16)] = z16
    return _
  lax.fori_loop(0, K, body, None)


def _zero_acc(rows, acc, sid, sem):
  del sem
  for t in range(RPS // K):
    pltpu.sync_copy(rows, acc.at[pl.ds(sid * RPS + t * K, K)])


@functools.partial(
    pl.kernel,
    out_type=jax.ShapeDtypeStruct((NC, NPAD, D), jnp.float32),
    mesh=_MESH,
    compiler_params=_SC_PARAMS,
    scratch_types=[
        pltpu.VMEM((PPC, K), jnp.int32),
        pltpu.VMEM((PPC, K), jnp.int32),
        pltpu.VMEM((K, D), jnp.float32),
        pltpu.VMEM((K, D), jnp.float32),
        pltpu.VMEM_SHARED((NPAD, D), jnp.float32),
        pltpu.SemaphoreType.DMA,
        pltpu.SemaphoreType.DMA,
        pltpu.SemaphoreType.DMA,
        pltpu.SemaphoreType.DMA,
    ],
)
def _seg_sum(x_hbm, idxin_hbm, idxout_hbm, out_hbm,
             idxin_v, idxout_v, rows_a, rows_b, acc,
             sem_ga, sem_gb, sem_sa, sem_sb):
  cid = lax.axis_index("c")
  sid = lax.axis_index("s")

  _zero_rows(rows_a)
  _zero_acc(rows_a, acc, sid, sem_sa)
  plsc.subcore_barrier()

  def gath(j, buf, sem):
    return pltpu.make_async_copy(x_hbm.at[idxin_v.at[j]], buf, sem)

  def scat(j, buf, sem):
    return pltpu.make_async_copy(buf, acc.at[idxout_v.at[j]], sem)

  npass = jnp.where(cid == 0, FAST_P, SLOW_P)
  base = jnp.where(cid == 0, sid * FAST_CPT, NS * FAST_CPT + sid * SLOW_CPT)

  def pass_body(p, _):
    off = pl.multiple_of(base + p * PPC, 8)
    pltpu.sync_copy(idxin_hbm.at[pl.ds(off, PPC)], idxin_v)
    pltpu.sync_copy(idxout_hbm.at[pl.ds(off, PPC)], idxout_v)

    gath(0, rows_a, sem_ga).start()
    gath(1, rows_b, sem_gb).start()

    def body(t, _):
      j0 = 2 * t
      j1 = 2 * t + 1
      gath(j0, rows_a, sem_ga).wait()
      scat(j0, rows_a, sem_sa).start(add=True)
      gath(j1, rows_b, sem_gb).wait()
      scat(j1, rows_b, sem_sb).start(add=True)
      scat(j0, rows_a, sem_sa).wait()
      gath(j0 + 2, rows_a, sem_ga).start()
      scat(j1, rows_b, sem_sb).wait()
      gath(j1 + 2, rows_b, sem_gb).start()
      return _
    lax.fori_loop(0, PPC // 2 - 1, body, None)

    gath(PPC - 2, rows_a, sem_ga).wait()
    scat(PPC - 2, rows_a, sem_sa).start(add=True)
    gath(PPC - 1, rows_b, sem_gb).wait()
    scat(PPC - 1, rows_b, sem_sb).start(add=True)
    scat(PPC - 2, rows_a, sem_sa).wait()
    scat(PPC - 1, rows_b, sem_sb).wait()
    return _
  lax.fori_loop(0, npass, pass_body, None)

  plsc.subcore_barrier()
  pltpu.sync_copy(acc.at[pl.ds(sid * RPS, RPS)],
                  out_hbm.at[cid, pl.ds(sid * RPS, RPS)])


@functools.partial(
    pl.kernel,
    out_type=(jax.ShapeDtypeStruct((NC, NPAD, D), jnp.float32),
              jax.ShapeDtypeStruct((NC, NPAD), jnp.float32)),
    mesh=_MESH,
    compiler_params=_SC_PARAMS,
    scratch_types=[
        pltpu.VMEM((PPC, K), jnp.int32),
        pltpu.VMEM((PPC, K), jnp.int32),
        pltpu.VMEM((NPAD // 128, 128), jnp.float32),
        pltpu.VMEM((NPAD // 128, 128), jnp.float32),
        pltpu.VMEM((16,), jnp.float32),
        pltpu.VMEM((1, K), jnp.float32),
        pltpu.VMEM((K, D), jnp.float32),
        pltpu.VMEM_SHARED((NPAD, D), jnp.float32),
        pltpu.VMEM_SHARED((NPAD,), jnp.float32),
        pltpu.SemaphoreType.DMA,
        pltpu.SemaphoreType.DMA,
    ],
)
def _gat_edges(src_hbm, dst_hbm, asrc_hbm, adst_hbm, m_hbm, xp_hbm,
               numer_hbm, denom_hbm,
               srcv, dstv, asv, adv, mv, exv, rows,
               nacc, dacc, sem, sem2):
  cid = lax.axis_index("c")
  sid = lax.axis_index("s")

  z16 = jnp.zeros((16,), jnp.float32)
  _zero_rows(rows)
  for t in range(K // 16):
    exv[0, pl.ds(t * 16, 16)] = z16

  _zero_acc(rows, nacc, sid, sem)
  for t in range(RPS // K):
    pltpu.sync_copy(exv.at[0], dacc.at[pl.ds(sid * RPS + t * K, K)])

  pltpu.sync_copy(asrc_hbm, asv)
  pltpu.sync_copy(adst_hbm, adv)
  pltpu.sync_copy(m_hbm, mv)
  plsc.subcore_barrier()

  z16i = jnp.zeros((16,), jnp.int32)

  def body(j, _):
    cp = pltpu.async_copy(xp_hbm.at[srcv.at[j]], rows, sem)
    mvec = mv[...]
    for k in range(K // 16):
      sidx = srcv[j, pl.ds(k * 16, 16)]
      didx = dstv[j, pl.ds(k * 16, 16)]
      s = (plsc.load_gather(asv, [sidx >> 7, sidx & 127]) +
           plsc.load_gather(adv, [didx >> 7, didx & 127]))
      e = jnp.where(s >= 0, s, 0.2 * s)
      exv[0, pl.ds(k * 16, 16)] = jnp.exp(e - mvec)
    pltpu.sync_copy(exv.at[0], dacc.at[dstv.at[j]], add=True)
    cp.wait()

    def scale_body(r, _):
      sc = plsc.load_gather(exv, [z16i, jnp.broadcast_to(r, (16,))])
      for k2 in range(D // 16):
        rows[r, pl.ds(k2 * 16, 16)] = rows[r, pl.ds(k2 * 16, 16)] * sc
      return _
    lax.fori_loop(0, K, scale_body, None)

    pltpu.sync_copy(rows, nacc.at[dstv.at[j]], add=True)
    return _

  npass = jnp.where(cid == 0, FAST_P, SLOW_P)
  base = jnp.where(cid == 0, sid * FAST_CPT, NS * FAST_CPT + sid * SLOW_CPT)

  def pass_body(p, _):
    off = pl.multiple_of(base + p * PPC, 8)
    pltpu.sync_copy(src_hbm.at[pl.ds(off, PPC)], srcv)
    pltpu.sync_copy(dst_hbm.at[pl.ds(off, PPC)], dstv)
    lax.fori_loop(0, PPC, body, None)
    return _
  lax.fori_loop(0, npass, pass_body, None)

  plsc.subcore_barrier()
  pltpu.sync_copy(nacc.at[pl.ds(sid * RPS, RPS)],
                  numer_hbm.at[cid, pl.ds(sid * RPS, RPS)])
  pltpu.sync_copy(dacc.at[pl.ds(sid * RPS, RPS)],
                  denom_hbm.at[cid, pl.ds(sid * RPS, RPS)])


BLK = 1000
GRID = N // BLK


def _mm_t(a, w):
  return lax.dot_general(a, w, (((1,), (1,)), ((), ())),
                         preferred_element_type=jnp.float32)


def _mlp_body(hp_ref, w1, b1, w2, b2, w3, b3, h3_ref, s_ref, sq_ref):
  i = pl.program_id(0)
  hb = hp_ref[0] + hp_ref[1]
  h1 = jnp.maximum(_mm_t(hb, w1[...]) + b1[...], 0.0)
  h2 = jnp.maximum(_mm_t(h1, w2[...]) + b2[...], 0.0)
  h3 = _mm_t(h2, w3[...]) + b3[...]
  h3_ref[...] = h3

  @pl.when(i == 0)
  def _():
    s_ref[...] = jnp.zeros_like(s_ref)
    sq_ref[...] = jnp.zeros_like(sq_ref)
  s_ref[...] += jnp.sum(h3, axis=0, keepdims=True)
  sq_ref[...] += jnp.sum(h3 * h3, axis=0, keepdims=True)


def _bn_gat_body(h3_ref, s_ref, sq_ref, gamma, beta, wg, att_s, att_d,
                 xp_ref, as_ref, ad_ref, ms_ref, md_ref, m_ref):
  i = pl.program_id(0)
  mean = s_ref[...] / N
  var = sq_ref[...] / N - mean * mean
  hn = (h3_ref[...] - mean) * lax.rsqrt(var + 1e-5) * gamma[...] + beta[...]
  xp = _mm_t(hn, wg[...])
  xp_ref[...] = xp
  a_s = jnp.sum(xp * att_s[...], axis=1)
  a_d = jnp.sum(xp * att_d[...], axis=1)
  as_ref[pl.ds(i, 1), :] = a_s[None, :]
  ad_ref[pl.ds(i, 1), :] = a_d[None, :]

  @pl.when(i == 0)
  def _():
    ms_ref[...] = jnp.full_like(ms_ref, -3.0e38)
    md_ref[...] = jnp.full_like(md_ref, -3.0e38)
  ms_ref[...] = jnp.maximum(ms_ref[...], jnp.max(a_s))
  md_ref[...] = jnp.maximum(md_ref[...], jnp.max(a_d))

  @pl.when(i == GRID - 1)
  def _():
    tot = ms_ref[0, 0] + md_ref[0, 0]
    m = jnp.where(tot >= 0, tot, 0.2 * tot)
    m_ref[...] = jnp.broadcast_to(m, m_ref.shape)


def _gat_combine_body(n_ref, d_ref, bias, out_ref):
  num = n_ref[0] + n_ref[1]
  den = d_ref[0] + d_ref[1]
  out_ref[...] = num / (den + 1e-16) + bias[...]


def _add_partials_body(p_ref, out_ref):
  out_ref[...] = p_ref[0] + p_ref[1]


def _pad_idx(idx, fill):
  pad = jnp.full((EPAD - E,), fill, dtype=jnp.int32)
  return jnp.concatenate([idx, pad]).reshape(CHUNKS, K)


def kernel(x, edge_index, pv_rows, pv_cols, W1, b1, W2, b2, W3, b3,
           gamma, beta, W_gat, att_src, att_dst, bias_gat):
  row = lambda v: v.reshape(1, D)
  full128 = pl.BlockSpec((D, D), lambda i: (0, 0))
  fullrow = pl.BlockSpec((1, D), lambda i: (0, 0))
  hpart_spec = pl.BlockSpec((NC, BLK, D), lambda i: (0, i, 0))
  blk_spec = pl.BlockSpec((BLK, D), lambda i: (i, 0))

  hpart = _seg_sum(x, _pad_idx(pv_cols, 0), _pad_idx(pv_rows, N))

  h3, ssum, ssq = pl.pallas_call(
      _mlp_body,
      grid=(GRID,),
      in_specs=[hpart_spec, full128, fullrow, full128, fullrow, full128,
                fullrow],
      out_specs=[blk_spec, fullrow, fullrow],
      out_shape=[jax.ShapeDtypeStruct((N, D), jnp.float32),
                 jax.ShapeDtypeStruct((1, D), jnp.float32),
                 jax.ShapeDtypeStruct((1, D), jnp.float32)],
  )(hpart, W1, row(b1), W2, row(b2), W3, row(b3))

  arow_spec = pl.BlockSpec((GRID, BLK), lambda i: (0, 0))
  xp, a_s, a_d, _, _, mrow = pl.pallas_call(
      _bn_gat_body,
      grid=(GRID,),
      in_specs=[blk_spec, fullrow, fullrow, fullrow, fullrow, full128,
                fullrow, fullrow],
      out_specs=[blk_spec, arow_spec, arow_spec, fullrow, fullrow, fullrow],
      out_shape=[jax.ShapeDtypeStruct((N, D), jnp.float32),
                 jax.ShapeDtypeStruct((GRID, BLK), jnp.float32),
                 jax.ShapeDtypeStruct((GRID, BLK), jnp.float32),
                 jax.ShapeDtypeStruct((1, D), jnp.float32),
                 jax.ShapeDtypeStruct((1, D), jnp.float32),
                 jax.ShapeDtypeStruct((1, D), jnp.float32)],
  )(h3, ssum, ssq, row(gamma), row(beta), W_gat, row(att_src), row(att_dst))

  zpad = jnp.zeros((NPAD - N,), jnp.float32)
  asrc_p = jnp.concatenate([a_s.reshape(N), zpad]).reshape(NPAD // 128, 128)
  adst_p = jnp.concatenate([a_d.reshape(N), zpad]).reshape(NPAD // 128, 128)

  numer, denom = _gat_edges(
      _pad_idx(edge_index[0], 0), _pad_idx(edge_index[1], N),
      asrc_p, adst_p, mrow[0, :16], xp)

  y = pl.pallas_call(
      _gat_combine_body,
      grid=(GRID,),
      in_specs=[hpart_spec, pl.BlockSpec((NC, BLK, 1), lambda i: (0, i, 0)),
                fullrow],
      out_specs=blk_spec,
      out_shape=jax.ShapeDtypeStruct((N, D), jnp.float32),
  )(numer, denom.reshape(NC, NPAD, 1), row(bias_gat))

  opart = _seg_sum(y, _pad_idx(pv_rows, 0), _pad_idx(pv_cols, N))

  out = pl.pallas_call(
      _add_partials_body,
      grid=(GRID,),
      in_specs=[hpart_spec],
      out_specs=blk_spec,
      out_shape=jax.ShapeDtypeStruct((N, D), jnp.float32),
  )(opart)
  return out


# speedup vs baseline: 1.4823x; 1.4823x over previous
import functools

import jax
import jax.numpy as jnp
from jax import lax
from jax.experimental import pallas as pl
from jax.experimental.pallas import tpu as pltpu
from jax.experimental.pallas import tpu_sc as plsc

NC = 2
NS = 16
NW = NC * NS

N = 10000
E = 320000
D = 128

NPAD = 10240
RPS = NPAD // NS
K = 128
CHUNKS = 2560
EPAD = CHUNKS * K
PPC = 32
FAST_P = 4
SLOW_P = 1
FAST_CPT = FAST_P * PPC
SLOW_CPT = SLOW_P * PPC
assert NS * (FAST_CPT + SLOW_CPT) == CHUNKS

_MESH = plsc.VectorSubcoreMesh(
    core_axis_name="c", subcore_axis_name="s", num_cores=NC, num_subcores=NS)
_SC_PARAMS = pltpu.CompilerParams(needs_layout_passes=False)


def _zero_rows(rows):
  z16 = jnp.zeros((16,), jnp.float32)

  def body(r, _):
    for k in range(D // 16):
      rows[r, pl.ds(k * 16, 16)] = z16
    return _
  lax.fori_loop(0, K, body, None)


def _zero_acc(rows, acc, sid, sem):
  del sem
  for t in range(RPS // K):
    pltpu.sync_copy(rows, acc.at[pl.ds(sid * RPS + t * K, K)])


@functools.partial(
    pl.kernel,
    out_type=jax.ShapeDtypeStruct((NC, NPAD, D), jnp.float32),
    mesh=_MESH,
    compiler_params=_SC_PARAMS,
    scratch_types=[
        pltpu.VMEM((PPC, K), jnp.int32),
        pltpu.VMEM((PPC, K), jnp.int32),
        pltpu.VMEM((K, D), jnp.float32),
        pltpu.VMEM((K, D), jnp.float32),
        pltpu.VMEM_SHARED((NPAD, D), jnp.float32),
        pltpu.SemaphoreType.DMA,
        pltpu.SemaphoreType.DMA,
        pltpu.SemaphoreType.DMA,
        pltpu.SemaphoreType.DMA,
    ],
)
def _seg_sum(x_hbm, idxin_hbm, idxout_hbm, out_hbm,
             idxin_v, idxout_v, rows_a, rows_b, acc,
             sem_ga, sem_gb, sem_sa, sem_sb):
  cid = lax.axis_index("c")
  sid = lax.axis_index("s")

  _zero_rows(rows_a)
  _zero_acc(rows_a, acc, sid, sem_sa)
  plsc.subcore_barrier()

  def gath(j, buf, sem):
    return pltpu.make_async_copy(x_hbm.at[idxin_v.at[j]], buf, sem)

  def scat(j, buf, sem):
    return pltpu.make_async_copy(buf, acc.at[idxout_v.at[j]], sem)

  npass = jnp.where(cid == 0, FAST_P, SLOW_P)
  base = jnp.where(cid == 0, sid * FAST_CPT, NS * FAST_CPT + sid * SLOW_CPT)

  def pass_body(p, _):
    off = pl.multiple_of(base + p * PPC, 8)
    pltpu.sync_copy(idxin_hbm.at[pl.ds(off, PPC)], idxin_v)
    pltpu.sync_copy(idxout_hbm.at[pl.ds(off, PPC)], idxout_v)

    gath(0, rows_a, sem_ga).start()
    gath(1, rows_b, sem_gb).start()

    def body(t, _):
      j0 = 2 * t
      j1 = 2 * t + 1
      gath(j0, rows_a, sem_ga).wait()
      scat(j0, rows_a, sem_sa).start(add=True)
      gath(j1, rows_b, sem_gb).wait()
      scat(j1, rows_b, sem_sb).start(add=True)
      scat(j0, rows_a, sem_sa).wait()
      gath(j0 + 2, rows_a, sem_ga).start()
      scat(j1, rows_b, sem_sb).wait()
      gath(j1 + 2, rows_b, sem_gb).start()
      return _
    lax.fori_loop(0, PPC // 2 - 1, body, None)

    gath(PPC - 2, rows_a, sem_ga).wait()
    scat(PPC - 2, rows_a, sem_sa).start(add=True)
    gath(PPC - 1, rows_b, sem_gb).wait()
    scat(PPC - 1, rows_b, sem_sb).start(add=True)
    scat(PPC - 2, rows_a, sem_sa).wait()
    scat(PPC - 1, rows_b, sem_sb).wait()
    return _
  lax.fori_loop(0, npass, pass_body, None)

  plsc.subcore_barrier()
  pltpu.sync_copy(acc.at[pl.ds(sid * RPS, RPS)],
                  out_hbm.at[cid, pl.ds(sid * RPS, RPS)])


@functools.partial(
    pl.kernel,
    out_type=(jax.ShapeDtypeStruct((NC, NPAD, D), jnp.float32),
              jax.ShapeDtypeStruct((NC, NPAD), jnp.float32)),
    mesh=_MESH,
    compiler_params=_SC_PARAMS,
    scratch_types=[
        pltpu.VMEM((PPC, K), jnp.int32),
        pltpu.VMEM((PPC, K), jnp.int32),
        pltpu.VMEM((NPAD // 128, 128), jnp.float32),
        pltpu.VMEM((NPAD // 128, 128), jnp.float32),
        pltpu.VMEM((16,), jnp.float32),
        pltpu.VMEM((1, K), jnp.float32),
        pltpu.VMEM((K, D), jnp.float32),
        pltpu.VMEM_SHARED((NPAD, D), jnp.float32),
        pltpu.VMEM_SHARED((NPAD,), jnp.float32),
        pltpu.SemaphoreType.DMA,
        pltpu.SemaphoreType.DMA,
    ],
)
def _gat_edges(src_hbm, dst_hbm, asrc_hbm, adst_hbm, m_hbm, xp_hbm,
               numer_hbm, denom_hbm,
               srcv, dstv, asv, adv, mv, exv, rows,
               nacc, dacc, sem, sem2):
  cid = lax.axis_index("c")
  sid = lax.axis_index("s")

  z16 = jnp.zeros((16,), jnp.float32)
  _zero_rows(rows)
  for t in range(K // 16):
    exv[0, pl.ds(t * 16, 16)] = z16

  _zero_acc(rows, nacc, sid, sem)
  for t in range(RPS // K):
    pltpu.sync_copy(exv.at[0], dacc.at[pl.ds(sid * RPS + t * K, K)])

  pltpu.sync_copy(asrc_hbm, asv)
  pltpu.sync_copy(adst_hbm, adv)
  pltpu.sync_copy(m_hbm, mv)
  plsc.subcore_barrier()

  z16i = jnp.zeros((16,), jnp.int32)

  def body(j, _):
    cp = pltpu.async_copy(xp_hbm.at[srcv.at[j]], rows, sem)
    mvec = mv[...]
    for k in range(K // 16):
      sidx = srcv[j, pl.ds(k * 16, 16)]
      didx = dstv[j, pl.ds(k * 16, 16)]
      s = (plsc.load_gather(asv, [sidx >> 7, sidx & 127]) +
           plsc.load_gather(adv, [didx >> 7, didx & 127]))
      e = jnp.where(s >= 0, s, 0.2 * s)
      exv[0, pl.ds(k * 16, 16)] = jnp.exp(e - mvec)
    pltpu.sync_copy(exv.at[0], dacc.at[dstv.at[j]], add=True)
    cp.wait()

    def scale_body(r, _):
      sc = plsc.load_gather(exv, [z16i, jnp.broadcast_to(r, (16,))])
      for k2 in range(D // 16):
        rows[r, pl.ds(k2 * 16, 16)] = rows[r, pl.ds(k2 * 16, 16)] * sc
      return _
    lax.fori_loop(0, K, scale_body, None)

    pltpu.sync_copy(rows, nacc.at[dstv.at[j]], add=True)
    return _

  npass = jnp.where(cid == 0, FAST_P, SLOW_P)
  base = jnp.where(cid == 0, sid * FAST_CPT, NS * FAST_CPT + sid * SLOW_CPT)

  def pass_body(p, _):
    off = pl.multiple_of(base + p * PPC, 8)
    pltpu.sync_copy(src_hbm.at[pl.ds(off, PPC)], srcv)
    pltpu.sync_copy(dst_hbm.at[pl.ds(off, PPC)], dstv)
    lax.fori_loop(0, PPC, body, None)
    return _
  lax.fori_loop(0, npass, pass_body, None)

  plsc.subcore_barrier()
  pltpu.sync_copy(nacc.at[pl.ds(sid * RPS, RPS)],
                  numer_hbm.at[cid, pl.ds(sid * RPS, RPS)])
  pltpu.sync_copy(dacc.at[pl.ds(sid * RPS, RPS)],
                  denom_hbm.at[cid, pl.ds(sid * RPS, RPS)])


BLK = 1000
GRID = N // BLK


def _mm_t(a, w):
  return lax.dot_general(a, w, (((1,), (1,)), ((), ())),
                         preferred_element_type=jnp.float32)


def _mlp_body(hp_ref, w1, b1, w2, b2, w3, b3, h3_ref, s_ref, sq_ref):
  i = pl.program_id(0)
  hb = hp_ref[0] + hp_ref[1]
  h1 = jnp.maximum(_mm_t(hb, w1[...]) + b1[...], 0.0)
  h2 = jnp.maximum(_mm_t(h1, w2[...]) + b2[...], 0.0)
  h3 = _mm_t(h2, w3[...]) + b3[...]
  h3_ref[...] = h3

  @pl.when(i == 0)
  def _():
    s_ref[...] = jnp.zeros_like(s_ref)
    sq_ref[...] = jnp.zeros_like(sq_ref)
  s_ref[...] += jnp.sum(h3, axis=0, keepdims=True)
  sq_ref[...] += jnp.sum(h3 * h3, axis=0, keepdims=True)


def _bn_gat_body(h3_ref, s_ref, sq_ref, gamma, beta, wg, att_s, att_d,
                 xp_ref, as_ref, ad_ref, ms_ref, md_ref, m_ref):
  i = pl.program_id(0)
  mean = s_ref[...] / N
  var = sq_ref[...] / N - mean * mean
  hn = (h3_ref[...] - mean) * lax.rsqrt(var + 1e-5) * gamma[...] + beta[...]
  xp = _mm_t(hn, wg[...])
  xp_ref[...] = xp
  a_s = jnp.sum(xp * att_s[...], axis=1)
  a_d = jnp.sum(xp * att_d[...], axis=1)
  as_ref[pl.ds(i, 1), :] = a_s[None, :]
  ad_ref[pl.ds(i, 1), :] = a_d[None, :]

  @pl.when(i == 0)
  def _():
    ms_ref[...] = jnp.full_like(ms_ref, -3.0e38)
    md_ref[...] = jnp.full_like(md_ref, -3.0e38)
  ms_ref[...] = jnp.maximum(ms_ref[...], jnp.max(a_s))
  md_ref[...] = jnp.maximum(md_ref[...], jnp.max(a_d))

  @pl.when(i == GRID - 1)
  def _():
    tot = ms_ref[0, 0] + md_ref[0, 0]
    m = jnp.where(tot >= 0, tot, 0.2 * tot)
    m_ref[...] = jnp.broadcast_to(m, m_ref.shape)


def _gat_combine_body(n_ref, d_ref, bias, out_ref):
  num = n_ref[0] + n_ref[1]
  den = d_ref[0] + d_ref[1]
  out_ref[...] = num / (den + 1e-16) + bias[...]


def _add_partials_body(p_ref, out_ref):
  out_ref[...] = p_ref[0] + p_ref[1]


def _pad_idx(idx, fill):
  pad = jnp.full((EPAD - E,), fill, dtype=jnp.int32)
  return jnp.concatenate([idx, pad]).reshape(CHUNKS, K)


def kernel(x, edge_index, pv_rows, pv_cols, W1, b1, W2, b2, W3, b3,
           gamma, beta, W_gat, att_src, att_dst, bias_gat):
  row = lambda v: v.reshape(1, D)
  full128 = pl.BlockSpec((D, D), lambda i: (0, 0))
  fullrow = pl.BlockSpec((1, D), lambda i: (0, 0))
  hpart_spec = pl.BlockSpec((NC, BLK, D), lambda i: (0, i, 0))
  blk_spec = pl.BlockSpec((BLK, D), lambda i: (i, 0))

  hpart = _seg_sum(x, _pad_idx(pv_cols, 0), _pad_idx(pv_rows, N))

  h3, ssum, ssq = pl.pallas_call(
      _mlp_body,
      grid=(GRID,),
      in_specs=[hpart_spec, full128, fullrow, full128, fullrow, full128,
                fullrow],
      out_specs=[blk_spec, fullrow, fullrow],
      out_shape=[jax.ShapeDtypeStruct((N, D), jnp.float32),
                 jax.ShapeDtypeStruct((1, D), jnp.float32),
                 jax.ShapeDtypeStruct((1, D), jnp.float32)],
  )(hpart, W1, row(b1), W2, row(b2), W3, row(b3))

  arow_spec = pl.BlockSpec((GRID, BLK), lambda i: (0, 0))
  xp, a_s, a_d, _, _, mrow = pl.pallas_call(
      _bn_gat_body,
      grid=(GRID,),
      in_specs=[blk_spec, fullrow, fullrow, fullrow, fullrow, full128,
                fullrow, fullrow],
      out_specs=[blk_spec, arow_spec, arow_spec, fullrow, fullrow, fullrow],
      out_shape=[jax.ShapeDtypeStruct((N, D), jnp.float32),
                 jax.ShapeDtypeStruct((GRID, BLK), jnp.float32),
                 jax.ShapeDtypeStruct((GRID, BLK), jnp.float32),
                 jax.ShapeDtypeStruct((1, D), jnp.float32),
                 jax.ShapeDtypeStruct((1, D), jnp.float32),
                 jax.ShapeDtypeStruct((1, D), jnp.float32)],
  )(h3, ssum, ssq, row(gamma), row(beta), W_gat, row(att_src), row(att_dst))

  zpad = jnp.zeros((NPAD - N,), jnp.float32)
  asrc_p = jnp.concatenate([a_s.reshape(N), zpad]).reshape(NPAD // 128, 128)
  adst_p = jnp.concatenate([a_d.reshape(N), zpad]).reshape(NPAD // 128, 128)

  numer, denom = _gat_edges(
      _pad_idx(edge_index[0], 0), _pad_idx(edge_index[1], N),
      asrc_p, adst_p, mrow[0, :16], xp)

  y = pl.pallas_call(
      _gat_combine_body,
      grid=(GRID,),
      in_specs=[hpart_spec, pl.BlockSpec((NC, BLK, 1), lambda i: (0, i, 0)),
                fullrow],
      out_specs=blk_spec,
      out_shape=jax.ShapeDtypeStruct((N, D), jnp.float32),
  )(numer, denom.reshape(NC, NPAD, 1), row(bias_gat))

  opart = _seg_sum(y, _pad_idx(pv_rows, 0), _pad_idx(pv_cols, N))

  out = pl.pallas_call(
      _add_partials_body,
      grid=(GRID,),
      in_specs=[hpart_spec],
      out_specs=blk_spec,
      out_shape=jax.ShapeDtypeStruct((N, D), jnp.float32),
  )(opart)
  return out
